# Initial kernel scaffold; baseline (speedup 1.0000x reference)
import jax
import jax.numpy as jnp
from jax.experimental import pallas as pl


def kernel(logits):
    raise NotImplementedError("write your pallas kernel here")


# speedup vs baseline: 1.0962x; 1.0962x over previous
import jax
import jax.numpy as jnp
from jax import lax
from jax.experimental import pallas as pl
from jax.experimental.pallas import tpu as pltpu
from jax.experimental.pallas import tpu_sc as plsc

_ROWS = 32768
_E = 64
_K = 8
_NC = 2
_NS = 16
_NW = _NC * _NS
_RPW = _ROWS // _NW


def _router_body(x_hbm, w_hbm, i_hbm, x_v, w_v, i_v):
    wid = lax.axis_index("s") * _NC + lax.axis_index("c")
    base = wid * _RPW
    pltpu.sync_copy(x_hbm.at[pl.ds(base * _E, _RPW * _E)], x_v)

    lane = lax.iota(jnp.int32, 16)
    lo_mask = lane < _K

    def merge(a, b):
        ka, va = a
        kb, vb = b
        kbr = lax.rev(kb, (0,))
        vbr = lax.rev(vb, (0,))
        take_a = ka >= kbr
        mk = jnp.where(take_a, ka, kbr)
        mv = jnp.where(take_a, va, vbr)
        return plsc.sort_key_val(mk, mv, descending=True)

    def row(r, carry):
        off = r * _E
        xs = [x_v[pl.ds(off + 16 * g, 16)] for g in range(4)]
        m = jnp.max(jnp.maximum(jnp.maximum(xs[0], xs[1]),
                                jnp.maximum(xs[2], xs[3])))
        s = jnp.sum(jnp.exp(xs[0] - m) + jnp.exp(xs[1] - m)
                    + jnp.exp(xs[2] - m) + jnp.exp(xs[3] - m))
        groups = [plsc.sort_key_val(xs[g], lane + 16 * g, descending=True)
                  for g in range(4)]
        fk, fv = merge(merge(groups[0], groups[1]),
                       merge(groups[2], groups[3]))
        w = jnp.exp(fk - m) / s
        plsc.store_compressed(w_v.at[pl.ds(r * _K, 16)], w, mask=lo_mask)
        plsc.store_compressed(i_v.at[pl.ds(r * _K, 16)], fv, mask=lo_mask)
        return carry

    lax.fori_loop(0, _RPW, row, None)

    pltpu.sync_copy(w_v.at[pl.ds(0, _RPW * _K)],
                    w_hbm.at[pl.ds(base * _K, _RPW * _K)])
    pltpu.sync_copy(i_v.at[pl.ds(0, _RPW * _K)],
                    i_hbm.at[pl.ds(base * _K, _RPW * _K)])


def _make_router():
    mesh = plsc.VectorSubcoreMesh(core_axis_name="c", subcore_axis_name="s",
                                  num_cores=_NC, num_subcores=_NS)
    return pl.kernel(
        _router_body,
        out_type=[jax.ShapeDtypeStruct((_ROWS * _K,), jnp.float32),
                  jax.ShapeDtypeStruct((_ROWS * _K,), jnp.int32)],
        mesh=mesh,
        scratch_types=[pltpu.VMEM((_RPW * _E,), jnp.float32),
                       pltpu.VMEM((_RPW * _K + 16,), jnp.float32),
                       pltpu.VMEM((_RPW * _K + 16,), jnp.int32)],
        compiler_params=pltpu.CompilerParams(needs_layout_passes=False),
    )


@jax.jit
def kernel(logits):
    w, i = _make_router()(logits.reshape(-1))
    return w.reshape(_ROWS, _K), i.reshape(_ROWS, _K)


# speedup vs baseline: 1.4457x; 1.3188x over previous
import jax
import jax.numpy as jnp
from jax import lax
from jax.experimental import pallas as pl
from jax.experimental.pallas import tpu as pltpu
from jax.experimental.pallas import tpu_sc as plsc

_ROWS = 32768
_E = 64
_K = 8
_NC = 2
_NS = 16
_NW = _NC * _NS
_RPW = _ROWS // _NW


def _router_body(x_hbm, w_hbm, i_hbm, x_v, w_v, i_v):
    wid = lax.axis_index("s") * _NC + lax.axis_index("c")
    base = wid * _RPW
    pltpu.sync_copy(x_hbm.at[pl.ds(base * _E, _RPW * _E)], x_v)

    lane = lax.iota(jnp.int32, 16)
    lo_mask = lane < _K

    def merge(a, b):
        ka, va = a
        kb, vb = b
        kbr = lax.rev(kb, (0,))
        vbr = lax.rev(vb, (0,))
        take_a = ka >= kbr
        mk = jnp.where(take_a, ka, kbr)
        mv = jnp.where(take_a, va, vbr)
        return plsc.sort_key_val(mk, mv, descending=True)

    @plsc.parallel_loop(0, _RPW, unroll=4)
    def row(r):
        off = r * _E
        xs = [x_v[pl.ds(off + 16 * g, 16)] for g in range(4)]
        m = jnp.max(jnp.maximum(jnp.maximum(xs[0], xs[1]),
                                jnp.maximum(xs[2], xs[3])))
        s = jnp.sum(jnp.exp(xs[0] - m) + jnp.exp(xs[1] - m)
                    + jnp.exp(xs[2] - m) + jnp.exp(xs[3] - m))
        groups = [plsc.sort_key_val(xs[g], lane + 16 * g, descending=True)
                  for g in range(4)]
        fk, fv = merge(merge(groups[0], groups[1]),
                       merge(groups[2], groups[3]))
        w = jnp.exp(fk - m) / s
        plsc.store_compressed(w_v.at[pl.ds(r * _K, 16)], w, mask=lo_mask)
        plsc.store_compressed(i_v.at[pl.ds(r * _K, 16)], fv, mask=lo_mask)

    pltpu.sync_copy(w_v.at[pl.ds(0, _RPW * _K)],
                    w_hbm.at[pl.ds(base * _K, _RPW * _K)])
    pltpu.sync_copy(i_v.at[pl.ds(0, _RPW * _K)],
                    i_hbm.at[pl.ds(base * _K, _RPW * _K)])


def _make_router():
    mesh = plsc.VectorSubcoreMesh(core_axis_name="c", subcore_axis_name="s",
                                  num_cores=_NC, num_subcores=_NS)
    return pl.kernel(
        _router_body,
        out_type=[jax.ShapeDtypeStruct((_ROWS * _K,), jnp.float32),
                  jax.ShapeDtypeStruct((_ROWS * _K,), jnp.int32)],
        mesh=mesh,
        scratch_types=[pltpu.VMEM((_RPW * _E,), jnp.float32),
                       pltpu.VMEM((_RPW * _K + 16,), jnp.float32),
                       pltpu.VMEM((_RPW * _K + 16,), jnp.int32)],
        compiler_params=pltpu.CompilerParams(needs_layout_passes=False),
    )


@jax.jit
def kernel(logits):
    w, i = _make_router()(logits.reshape(-1))
    return w.reshape(_ROWS, _K), i.reshape(_ROWS, _K)
